# baseline (speedup 1.0000x reference)
import functools

import jax
import jax.numpy as jnp
from jax import lax
from jax.experimental import pallas as pl
from jax.experimental.pallas import tpu as pltpu
from jax.experimental.pallas import tpu_sc as plsc

N = 10000
E = 160000
D = 256
H = 128
NG = 256
NL = 3
FEAT = 78

BR = 512
NP = 10240
EP = 163840
EB = 128
NB = EP // (16 * EB)
NBD = EP // (32 * EB)
ROWS_T = NP // 16
NBUF = 2

_MESH = plsc.VectorSubcoreMesh(core_axis_name="c", subcore_axis_name="s")


def _segsum_body(v0, v1, src2, dst2, zer, o0, o1, srcs, dsts, r0, r1,
                 acc, s0, s1):
    rows = (r0, r1)
    sem = (s0, s1)
    c = lax.axis_index("c")
    s = lax.axis_index("s")
    base = s * ROWS_T
    pltpu.sync_copy(zer, acc.at[pl.ds(base, ROWS_T)])
    plsc.subcore_barrier()

    def run(vh, oh):
        for hlf in range(2):
            pltpu.sync_copy(
                src2.at[pl.ds(s * NB + hlf * (NB // 2), NB // 2)], srcs)
            pltpu.sync_copy(
                dst2.at[pl.ds(s * NB + hlf * (NB // 2), NB // 2)], dsts)

            def body(j, carry):
                pltpu.async_copy(vh.at[srcs.at[j]], rows[0], sem[0]).wait()
                pltpu.sync_copy(rows[0], acc.at[dsts.at[j]], add=True)
                return carry

            lax.fori_loop(0, NB // 2, body, 0)
        plsc.subcore_barrier()
        pltpu.sync_copy(acc.at[pl.ds(base, ROWS_T)], oh.at[pl.ds(base, ROWS_T)])

    @pl.when(c == 0)
    def _():
        run(v0, o0)

    @pl.when(c == 1)
    def _():
        run(v1, o1)


_segsum_call = pl.kernel(
    _segsum_body,
    out_type=(jax.ShapeDtypeStruct((NP, H), jnp.float32),) * 2,
    mesh=_MESH,
    scratch_types=[
        pltpu.VMEM((NB // 2, EB), jnp.int32),
        pltpu.VMEM((NB // 2, EB), jnp.int32),
        pltpu.VMEM((EB, H), jnp.float32),
        pltpu.VMEM((EB, H), jnp.float32),
        pltpu.VMEM_SHARED((NP, H), jnp.float32),
        pltpu.SemaphoreType.DMA,
        pltpu.SemaphoreType.DMA,
    ],
)


def _mm(a, b):
    return jnp.dot(a, b)


def _row_spec(width):
    return pl.BlockSpec((BR, width), lambda i: (i, 0))


def _full_spec(r, cdim):
    return pl.BlockSpec((r, cdim), lambda i: (0, 0))


def _dinv(dg_ref):
    return lax.rsqrt(dg_ref[...][:, :1] + 1.0)


def _embed_body(x_ref, emb_ref, w_ref, b_ref, o0_ref, o1_ref):
    x = x_ref[...]
    x44 = x[:, :44]
    mx = jnp.max(x44, axis=1, keepdims=True)
    iota = lax.broadcasted_iota(jnp.int32, x44.shape, 1)
    idx = jnp.min(jnp.where(x44 >= mx, iota, 44), axis=1, keepdims=True)
    onehot = (iota == idx).astype(jnp.float32)
    emb_sel = jnp.dot(onehot, emb_ref[...], precision=lax.Precision.HIGHEST)
    hcat = jnp.concatenate([emb_sel, x[:, 44:FEAT]], axis=1)
    h = jnp.maximum(_mm(hcat, w_ref[...]) + b_ref[...], 0.0)
    o0_ref[...] = h[:, :H]
    o1_ref[...] = h[:, H:]


def _embed(x_pad, emb, lin_W, lin_b):
    return pl.pallas_call(
        _embed_body,
        grid=(NP // BR,),
        in_specs=[_row_spec(128), _full_spec(44, 128), _full_spec(162, D),
                  _full_spec(1, D)],
        out_specs=(_row_spec(H), _row_spec(H)),
        out_shape=(jax.ShapeDtypeStruct((NP, H), jnp.float32),) * 2,
    )(x_pad, emb, lin_W, lin_b.reshape(1, D))


def _mm_scale_body(h0_ref, h1_ref, w_ref, dg_ref, o0_ref, o1_ref):
    h = jnp.concatenate([h0_ref[...], h1_ref[...]], axis=1)
    y = _mm(h, w_ref[...]) * _dinv(dg_ref)
    o0_ref[...] = y[:, :H]
    o1_ref[...] = y[:, H:]


def _mm_scale(h0, h1, W, dg):
    return pl.pallas_call(
        _mm_scale_body,
        grid=(NP // BR,),
        in_specs=[_row_spec(H), _row_spec(H), _full_spec(D, D),
                  _row_spec(H)],
        out_specs=(_row_spec(H), _row_spec(H)),
        out_shape=(jax.ShapeDtypeStruct((NP, H), jnp.float32),) * 2,
    )(h0, h1, W, dg)


def _gcn_post_body(a0_ref, a1_ref, y0_ref, y1_ref, dg_ref, b_ref,
                   o0_ref, o1_ref):
    agg = jnp.concatenate([a0_ref[...], a1_ref[...]], axis=1)
    y = jnp.concatenate([y0_ref[...], y1_ref[...]], axis=1)
    h = jnp.maximum((agg + y) * _dinv(dg_ref) + b_ref[...], 0.0)
    o0_ref[...] = h[:, :H]
    o1_ref[...] = h[:, H:]


def _gcn_post(a0, a1, y0, y1, dg, b):
    return pl.pallas_call(
        _gcn_post_body,
        grid=(NP // BR,),
        in_specs=[_row_spec(H)] * 4 + [_row_spec(H), _full_spec(1, D)],
        out_specs=(_row_spec(H), _row_spec(H)),
        out_shape=(jax.ShapeDtypeStruct((NP, H), jnp.float32),) * 2,
    )(a0, a1, y0, y1, dg, b.reshape(1, D))


def _layer_body(h0_ref, h1_ref, a0_ref, a1_ref, w1_ref, b1_ref, w2_ref, b2_ref,
                bng_ref, bnb_ref, wih_ref, bih_ref, whh_ref, bhh_ref,
                lng_ref, lnb_ref, m0_ref, m1_ref, e0_ref, e1_ref,
                ho0_ref, ho1_ref, mo0_ref, mo1_ref, eo0_ref, eo1_ref):
    h = jnp.concatenate([h0_ref[...], h1_ref[...]], axis=1)
    agg = jnp.concatenate([a0_ref[...], a1_ref[...]], axis=1)
    z = h + agg
    z = jnp.maximum(_mm(z, w1_ref[...]) + b1_ref[...], 0.0)
    z = _mm(z, w2_ref[...]) + b2_ref[...]
    z = jnp.maximum(z, 0.0) * bng_ref[...] + bnb_ref[...]
    gi = _mm(z, wih_ref[...]) + bih_ref[...]
    gh = _mm(h, whh_ref[...]) + bhh_ref[...]
    r = jax.nn.sigmoid(gi[:, :D] + gh[:, :D])
    zg = jax.nn.sigmoid(gi[:, D:2 * D] + gh[:, D:2 * D])
    n = jnp.tanh(gi[:, 2 * D:] + r * gh[:, 2 * D:])
    hq = (1.0 - zg) * n + zg * h
    mu = jnp.mean(hq, axis=1, keepdims=True)
    var = jnp.mean((hq - mu) ** 2, axis=1, keepdims=True)
    hn = (hq - mu) / jnp.sqrt(var + 1e-5) * lng_ref[...] + lnb_ref[...]
    m = jnp.concatenate([m0_ref[...], m1_ref[...]], axis=1) * hn
    e = jnp.concatenate([e0_ref[...], e1_ref[...]], axis=1) + hn
    ho0_ref[...] = hn[:, :H]
    ho1_ref[...] = hn[:, H:]
    mo0_ref[...] = m[:, :H]
    mo1_ref[...] = m[:, H:]
    eo0_ref[...] = e[:, :H]
    eo1_ref[...] = e[:, H:]


def _layer(h0, h1, a0, a1, m0, m1, e0, e1, p, i):
    row = _row_spec(H)
    return pl.pallas_call(
        _layer_body,
        grid=(NP // BR,),
        in_specs=[row, row, row, row, _full_spec(D, D), _full_spec(1, D),
                  _full_spec(D, D), _full_spec(1, D),
                  _full_spec(1, D), _full_spec(1, D),
                  _full_spec(D, 3 * D), _full_spec(1, 3 * D),
                  _full_spec(D, 3 * D), _full_spec(1, 3 * D),
                  _full_spec(1, D), _full_spec(1, D), row, row, row, row],
        out_specs=(row,) * 6,
        out_shape=(jax.ShapeDtypeStruct((NP, H), jnp.float32),) * 6,
    )(h0, h1, a0, a1, p[f'gin{i}_W1'], p[f'gin{i}_b1'].reshape(1, D),
      p[f'gin{i}_W2'], p[f'gin{i}_b2'].reshape(1, D),
      p[f'bn{i}_g'].reshape(1, D), p[f'bn{i}_b'].reshape(1, D),
      p[f'gru{i}_Wih'], p[f'gru{i}_bih'].reshape(1, 3 * D),
      p[f'gru{i}_Whh'], p[f'gru{i}_bhh'].reshape(1, 3 * D),
      p['ln_g'].reshape(1, D), p['ln_b'].reshape(1, D), m0, m1, e0, e1)


def kernel(x, edge_index, batch, params):
    p = params
    src, dst = edge_index[0], edge_index[1]
    pad = EP - E
    src2 = jnp.concatenate([src, jnp.zeros((pad,), jnp.int32)]).reshape(-1, EB)
    dst2 = jnp.concatenate([dst, jnp.full((pad,), NP - 1, jnp.int32)]
                           ).reshape(-1, EB)
    zer = jnp.zeros((ROWS_T, H), jnp.float32)
    ones_h = jnp.ones((NP, H), jnp.float32)

    segsum = lambda v0, v1: _segsum_call(v0, v1, src2, dst2, zer)

    x_pad = jnp.zeros((NP, 128), jnp.float32).at[:N, :FEAT].set(x)
    h0, h1 = _embed(x_pad, p['emb'], p['lin_W'], p['lin_b'])

    dg, _ = segsum(ones_h, ones_h)

    for wname, bname in (('gcn1_W', 'gcn1_b'), ('gcn2_W', 'gcn2_b')):
        y0, y1 = _mm_scale(h0, h1, p[wname], dg)
        a0, a1 = segsum(y0, y1)
        h0, h1 = _gcn_post(a0, a1, y0, y1, dg, p[bname])

    m0 = jnp.ones((NP, H), jnp.float32)
    m1 = jnp.ones((NP, H), jnp.float32)
    e0 = jnp.zeros((NP, H), jnp.float32)
    e1 = jnp.zeros((NP, H), jnp.float32)
    hs = []
    for i in range(NL):
        a0, a1 = segsum(h0, h1)
        h0, h1, m0, m1, e0, e1 = _layer(h0, h1, a0, a1, m0, m1, e0, e1, p, i)
        hs.append((h0, h1))

    parts = hs + [(m0, m1), (e0, e1)]
    node = jnp.concatenate([q for pr in parts for q in pr], axis=1)
    pooled = jax.ops.segment_max(node[:N], batch, num_segments=NG)
    return jnp.where(jnp.isfinite(pooled), pooled, 0.0)


# speedup vs baseline: 1.1859x; 1.1859x over previous
import functools

import jax
import jax.numpy as jnp
from jax import lax
from jax.experimental import pallas as pl
from jax.experimental.pallas import tpu as pltpu
from jax.experimental.pallas import tpu_sc as plsc

N = 10000
E = 160000
D = 256
H = 128
NG = 256
NL = 3
FEAT = 78

BR = 512
NP = 10240
EP = 163840
EB = 128
NB = EP // (16 * EB)
NBD = EP // (32 * EB)
ROWS_T = NP // 16
NBUF = 2

_MESH = plsc.VectorSubcoreMesh(core_axis_name="c", subcore_axis_name="s")


def _segsum_body(v0, v1, src2, dst2, zer, o0, o1, srcs, dsts, r0, r1,
                 acc, s0, s1):
    rows = (r0, r1)
    sem = (s0, s1)
    c = lax.axis_index("c")
    s = lax.axis_index("s")
    base = s * ROWS_T
    pltpu.sync_copy(zer, acc.at[pl.ds(base, ROWS_T)])
    plsc.subcore_barrier()

    def run(vh, oh):
        for hlf in range(2):
            pltpu.sync_copy(
                src2.at[pl.ds(s * NB + hlf * (NB // 2), NB // 2)], srcs)
            pltpu.sync_copy(
                dst2.at[pl.ds(s * NB + hlf * (NB // 2), NB // 2)], dsts)

            def body(j, carry):
                pltpu.async_copy(vh.at[srcs.at[j]], rows[0], sem[0]).wait()
                pltpu.sync_copy(rows[0], acc.at[dsts.at[j]], add=True)
                return carry

            lax.fori_loop(0, NB // 2, body, 0)
        plsc.subcore_barrier()
        pltpu.sync_copy(acc.at[pl.ds(base, ROWS_T)], oh.at[pl.ds(base, ROWS_T)])

    @pl.when(c == 0)
    def _():
        run(v0, o0)

    @pl.when(c == 1)
    def _():
        run(v1, o1)


_segsum_call = pl.kernel(
    _segsum_body,
    out_type=(jax.ShapeDtypeStruct((NP, H), jnp.float32),) * 2,
    mesh=_MESH,
    scratch_types=[
        pltpu.VMEM((NB // 2, EB), jnp.int32),
        pltpu.VMEM((NB // 2, EB), jnp.int32),
        pltpu.VMEM((EB, H), jnp.float32),
        pltpu.VMEM((EB, H), jnp.float32),
        pltpu.VMEM_SHARED((NP, H), jnp.float32),
        pltpu.SemaphoreType.DMA,
        pltpu.SemaphoreType.DMA,
    ],
)


def _deg_body(dst2, ones_hbm, zer, dga, dgb, dsts, rows, acc):
    c = lax.axis_index("c")
    s = lax.axis_index("s")
    base = s * ROWS_T
    pltpu.sync_copy(zer, acc.at[pl.ds(base, ROWS_T)])
    pltpu.sync_copy(ones_hbm, rows)
    plsc.subcore_barrier()

    def run(oh, off):
        pltpu.sync_copy(dst2.at[pl.ds(s * NB + off, NB // 2)], dsts)

        def body(j, carry):
            pltpu.sync_copy(rows, acc.at[dsts.at[j]], add=True)
            return carry

        lax.fori_loop(0, NB // 2, body, 0)
        plsc.subcore_barrier()
        pltpu.sync_copy(acc.at[pl.ds(base, ROWS_T)], oh.at[pl.ds(base, ROWS_T)])

    @pl.when(c == 0)
    def _():
        run(dga, 0)

    @pl.when(c == 1)
    def _():
        run(dgb, NB // 2)


_deg_call = pl.kernel(
    _deg_body,
    out_type=(jax.ShapeDtypeStruct((NP, H), jnp.float32),) * 2,
    mesh=_MESH,
    scratch_types=[
        pltpu.VMEM((NB // 2, EB), jnp.int32),
        pltpu.VMEM((EB, H), jnp.float32),
        pltpu.VMEM_SHARED((NP, H), jnp.float32),
    ],
)


def _mm(a, b):
    return jnp.dot(a, b)


def _row_spec(width):
    return pl.BlockSpec((BR, width), lambda i: (i, 0))


def _full_spec(r, cdim):
    return pl.BlockSpec((r, cdim), lambda i: (0, 0))


def _dinv(dga_ref, dgb_ref):
    return lax.rsqrt(dga_ref[...][:, :1] + dgb_ref[...][:, :1] + 1.0)


def _embed_body(x_ref, emb_ref, w_ref, b_ref, o0_ref, o1_ref):
    x = x_ref[...]
    x44 = x[:, :44]
    mx = jnp.max(x44, axis=1, keepdims=True)
    iota = lax.broadcasted_iota(jnp.int32, x44.shape, 1)
    idx = jnp.min(jnp.where(x44 >= mx, iota, 44), axis=1, keepdims=True)
    onehot = (iota == idx).astype(jnp.float32)
    emb_sel = jnp.dot(onehot, emb_ref[...], precision=lax.Precision.HIGHEST)
    hcat = jnp.concatenate([emb_sel, x[:, 44:FEAT]], axis=1)
    h = jnp.maximum(_mm(hcat, w_ref[...]) + b_ref[...], 0.0)
    o0_ref[...] = h[:, :H]
    o1_ref[...] = h[:, H:]


def _embed(x_pad, emb, lin_W, lin_b):
    return pl.pallas_call(
        _embed_body,
        grid=(NP // BR,),
        in_specs=[_row_spec(128), _full_spec(44, 128), _full_spec(162, D),
                  _full_spec(1, D)],
        out_specs=(_row_spec(H), _row_spec(H)),
        out_shape=(jax.ShapeDtypeStruct((NP, H), jnp.float32),) * 2,
    )(x_pad, emb, lin_W, lin_b.reshape(1, D))


def _mm_scale_body(h0_ref, h1_ref, w_ref, dga_ref, dgb_ref, o0_ref, o1_ref):
    h = jnp.concatenate([h0_ref[...], h1_ref[...]], axis=1)
    y = _mm(h, w_ref[...]) * _dinv(dga_ref, dgb_ref)
    o0_ref[...] = y[:, :H]
    o1_ref[...] = y[:, H:]


def _mm_scale(h0, h1, W, dga, dgb):
    return pl.pallas_call(
        _mm_scale_body,
        grid=(NP // BR,),
        in_specs=[_row_spec(H), _row_spec(H), _full_spec(D, D),
                  _row_spec(H), _row_spec(H)],
        out_specs=(_row_spec(H), _row_spec(H)),
        out_shape=(jax.ShapeDtypeStruct((NP, H), jnp.float32),) * 2,
    )(h0, h1, W, dga, dgb)


def _gcn_post_body(a0_ref, a1_ref, y0_ref, y1_ref, dga_ref, dgb_ref, b_ref,
                   o0_ref, o1_ref):
    agg = jnp.concatenate([a0_ref[...], a1_ref[...]], axis=1)
    y = jnp.concatenate([y0_ref[...], y1_ref[...]], axis=1)
    h = jnp.maximum((agg + y) * _dinv(dga_ref, dgb_ref) + b_ref[...], 0.0)
    o0_ref[...] = h[:, :H]
    o1_ref[...] = h[:, H:]


def _gcn_post(a0, a1, y0, y1, dga, dgb, b):
    return pl.pallas_call(
        _gcn_post_body,
        grid=(NP // BR,),
        in_specs=[_row_spec(H)] * 6 + [_full_spec(1, D)],
        out_specs=(_row_spec(H), _row_spec(H)),
        out_shape=(jax.ShapeDtypeStruct((NP, H), jnp.float32),) * 2,
    )(a0, a1, y0, y1, dga, dgb, b.reshape(1, D))


def _layer_body(h0_ref, h1_ref, a0_ref, a1_ref, w1_ref, b1_ref, w2_ref, b2_ref,
                bng_ref, bnb_ref, wih_ref, bih_ref, whh_ref, bhh_ref,
                lng_ref, lnb_ref, m0_ref, m1_ref, e0_ref, e1_ref,
                ho0_ref, ho1_ref, mo0_ref, mo1_ref, eo0_ref, eo1_ref):
    h = jnp.concatenate([h0_ref[...], h1_ref[...]], axis=1)
    agg = jnp.concatenate([a0_ref[...], a1_ref[...]], axis=1)
    z = h + agg
    z = jnp.maximum(_mm(z, w1_ref[...]) + b1_ref[...], 0.0)
    z = _mm(z, w2_ref[...]) + b2_ref[...]
    z = jnp.maximum(z, 0.0) * bng_ref[...] + bnb_ref[...]
    gi = _mm(z, wih_ref[...]) + bih_ref[...]
    gh = _mm(h, whh_ref[...]) + bhh_ref[...]
    r = jax.nn.sigmoid(gi[:, :D] + gh[:, :D])
    zg = jax.nn.sigmoid(gi[:, D:2 * D] + gh[:, D:2 * D])
    n = jnp.tanh(gi[:, 2 * D:] + r * gh[:, 2 * D:])
    hq = (1.0 - zg) * n + zg * h
    mu = jnp.mean(hq, axis=1, keepdims=True)
    var = jnp.mean((hq - mu) ** 2, axis=1, keepdims=True)
    hn = (hq - mu) / jnp.sqrt(var + 1e-5) * lng_ref[...] + lnb_ref[...]
    m = jnp.concatenate([m0_ref[...], m1_ref[...]], axis=1) * hn
    e = jnp.concatenate([e0_ref[...], e1_ref[...]], axis=1) + hn
    ho0_ref[...] = hn[:, :H]
    ho1_ref[...] = hn[:, H:]
    mo0_ref[...] = m[:, :H]
    mo1_ref[...] = m[:, H:]
    eo0_ref[...] = e[:, :H]
    eo1_ref[...] = e[:, H:]


def _layer(h0, h1, a0, a1, m0, m1, e0, e1, p, i):
    row = _row_spec(H)
    return pl.pallas_call(
        _layer_body,
        grid=(NP // BR,),
        in_specs=[row, row, row, row, _full_spec(D, D), _full_spec(1, D),
                  _full_spec(D, D), _full_spec(1, D),
                  _full_spec(1, D), _full_spec(1, D),
                  _full_spec(D, 3 * D), _full_spec(1, 3 * D),
                  _full_spec(D, 3 * D), _full_spec(1, 3 * D),
                  _full_spec(1, D), _full_spec(1, D), row, row, row, row],
        out_specs=(row,) * 6,
        out_shape=(jax.ShapeDtypeStruct((NP, H), jnp.float32),) * 6,
    )(h0, h1, a0, a1, p[f'gin{i}_W1'], p[f'gin{i}_b1'].reshape(1, D),
      p[f'gin{i}_W2'], p[f'gin{i}_b2'].reshape(1, D),
      p[f'bn{i}_g'].reshape(1, D), p[f'bn{i}_b'].reshape(1, D),
      p[f'gru{i}_Wih'], p[f'gru{i}_bih'].reshape(1, 3 * D),
      p[f'gru{i}_Whh'], p[f'gru{i}_bhh'].reshape(1, 3 * D),
      p['ln_g'].reshape(1, D), p['ln_b'].reshape(1, D), m0, m1, e0, e1)


def kernel(x, edge_index, batch, params):
    p = params
    src, dst = edge_index[0], edge_index[1]
    pad = EP - E
    src2 = jnp.concatenate([src, jnp.zeros((pad,), jnp.int32)]).reshape(-1, EB)
    dst2 = jnp.concatenate([dst, jnp.full((pad,), NP - 1, jnp.int32)]
                           ).reshape(-1, EB)
    zer = jnp.zeros((ROWS_T, H), jnp.float32)
    ones_eb = jnp.ones((EB, H), jnp.float32)

    segsum = lambda v0, v1: _segsum_call(v0, v1, src2, dst2, zer)

    x_pad = jnp.zeros((NP, 128), jnp.float32).at[:N, :FEAT].set(x)
    h0, h1 = _embed(x_pad, p['emb'], p['lin_W'], p['lin_b'])

    dga, dgb = _deg_call(dst2, ones_eb, zer)

    for wname, bname in (('gcn1_W', 'gcn1_b'), ('gcn2_W', 'gcn2_b')):
        y0, y1 = _mm_scale(h0, h1, p[wname], dga, dgb)
        a0, a1 = segsum(y0, y1)
        h0, h1 = _gcn_post(a0, a1, y0, y1, dga, dgb, p[bname])

    m0 = jnp.ones((NP, H), jnp.float32)
    m1 = jnp.ones((NP, H), jnp.float32)
    e0 = jnp.zeros((NP, H), jnp.float32)
    e1 = jnp.zeros((NP, H), jnp.float32)
    hs = []
    for i in range(NL):
        a0, a1 = segsum(h0, h1)
        h0, h1, m0, m1, e0, e1 = _layer(h0, h1, a0, a1, m0, m1, e0, e1, p, i)
        hs.append((h0, h1))

    parts = hs + [(m0, m1), (e0, e1)]
    node = jnp.concatenate([q for pr in parts for q in pr], axis=1)
    pooled = jax.ops.segment_max(node[:N], batch, num_segments=NG)
    return jnp.where(jnp.isfinite(pooled), pooled, 0.0)
